# baseline (speedup 1.0000x reference)
import functools

import jax
import jax.numpy as jnp
from jax import lax
from jax.experimental import pallas as pl
from jax.experimental.pallas import tpu as pltpu
from jax.experimental.pallas import tpu_sc as plsc

B = 4096
M = 200
D = 128
NC = 2
NS = 16
NW = NC * NS
BQ = B // NW
LANES = 16
DJ = D // LANES
MP = 208
MC = MP // LANES
MH0 = 104
MH1 = M - MH0


def _lane_perm(v, perm):
    return v.at[perm].get(mode="promise_in_bounds")


def _ges_kernel(qid_hbm, qcat_hbm, qbrand_hbm, match_hbm,
                id_tab, cat_tab, brand_tab, out_tab,
                out_hbm,
                qidx_v, midx_v, hid_v, t1_v, t2_v,
                rows0_v, rows1_v, logits0_v, logits1_v,
                sem0, sem1, semo0, semo1):
    wid = lax.axis_index("s") * NC + lax.axis_index("c")
    base = wid * BQ
    rows = [rows0_v, rows1_v]
    logits = [logits0_v, logits1_v]
    sems = [sem0, sem1]
    semos = [semo0, semo1]

    iota = lax.iota(jnp.int32, LANES)
    perms_p = [(iota + (1 << s)) & (LANES - 1) for s in range(4)]
    perms_m = [(iota + LANES - (1 << s)) & (LANES - 1) for s in range(4)]
    masks = [(iota % (2 << s)) < (1 << s) for s in range(4)]

    def _issue(q, slot):
        o0 = pl.multiple_of(q * M, 8)
        o1 = pl.multiple_of(q * M + MH0, 8)
        pltpu.async_copy(out_tab.at[midx_v.at[pl.ds(o0, MH0)]],
                         rows[slot].at[pl.ds(0, MH0)], sems[slot])
        pltpu.async_copy(out_tab.at[midx_v.at[pl.ds(o1, MH1)]],
                         rows[slot].at[pl.ds(MH0, MH1)], sems[slot])

    def _wait(q, slot):
        o0 = pl.multiple_of(q * M, 8)
        o1 = pl.multiple_of(q * M + MH0, 8)
        pltpu.make_async_copy(out_tab.at[midx_v.at[pl.ds(o0, MH0)]],
                              rows[slot].at[pl.ds(0, MH0)], sems[slot]).wait()
        pltpu.make_async_copy(out_tab.at[midx_v.at[pl.ds(o1, MH1)]],
                              rows[slot].at[pl.ds(MH0, MH1)], sems[slot]).wait()

    def _out_copy(q, slot):
        pltpu.async_copy(
            logits[slot].at[pl.ds(0, M)],
            out_hbm.at[pl.ds(pl.multiple_of((base + q) * M, 8), M)],
            semos[slot])

    def _out_wait(q, slot):
        pltpu.make_async_copy(
            logits[slot].at[pl.ds(0, M)],
            out_hbm.at[pl.ds(pl.multiple_of((base + q) * M, 8), M)],
            semos[slot]).wait()

    pltpu.sync_copy(match_hbm.at[pl.ds(base * M, BQ * M)], midx_v)
    pltpu.sync_copy(qid_hbm.at[pl.ds(base, BQ)], qidx_v.at[0])
    pltpu.sync_copy(qcat_hbm.at[pl.ds(base, BQ)], qidx_v.at[1])
    pltpu.sync_copy(qbrand_hbm.at[pl.ds(base, BQ)], qidx_v.at[2])
    cp0 = pltpu.async_copy(id_tab.at[qidx_v.at[0]], hid_v, semo0)
    cp1 = pltpu.async_copy(cat_tab.at[qidx_v.at[1]], t1_v, semo0)
    cp2 = pltpu.async_copy(brand_tab.at[qidx_v.at[2]], t2_v, semo0)
    _issue(0, 0)
    _issue(1, 1)
    cp0.wait()
    cp1.wait()
    cp2.wait()

    def _hid_body(i, _):
        q = i // DJ
        j = (i % DJ) * LANES
        hid_v[q, pl.ds(j, LANES)] = (
            hid_v[q, pl.ds(j, LANES)]
            + t1_v[q, pl.ds(j, LANES)]
            + t2_v[q, pl.ds(j, LANES)]
        ) * (1.0 / 3.0)
        return 0

    lax.fori_loop(0, BQ * DJ, _hid_body, 0)

    zeros = jnp.zeros((LANES,), jnp.float32)
    for rv in rows:
        for m in range(M, MP):
            for j in range(DJ):
                rv[m, pl.ds(j * LANES, LANES)] = zeros

    def _compute(q, slot):
        rv = rows[slot]
        hq = [hid_v[q, pl.ds(j * LANES, LANES)] for j in range(DJ)]
        @pl.when(q >= 2)
        def _():
            _out_wait(q - 2, slot)

        def _chunk(mc, _):
            mb = mc * LANES
            accs = []
            for i in range(LANES):
                a = rv[mb + i, pl.ds(0, LANES)] * hq[0]
                for j in range(1, DJ):
                    a = a + rv[mb + i, pl.ds(j * LANES, LANES)] * hq[j]
                accs.append(a)
            cur = accs
            for s in range(4):
                nxt = []
                for k in range(len(cur) // 2):
                    a0, a1 = cur[2 * k], cur[2 * k + 1]
                    x = a0 + _lane_perm(a0, perms_p[s])
                    y = a1 + _lane_perm(a1, perms_m[s])
                    nxt.append(jnp.where(masks[s], x, y))
                cur = nxt
            logits[slot][pl.ds(mb, LANES)] = cur[0]
            return 0

        lax.fori_loop(0, MC, _chunk, 0)
        _out_copy(q, slot)

    def _pair_body(i, _):
        q0 = 2 * i
        _wait(q0, 0)
        _compute(q0, 0)
        _issue(jnp.minimum(q0 + 2, BQ - 1), 0)
        _wait(q0 + 1, 1)
        _compute(q0 + 1, 1)
        _issue(jnp.minimum(q0 + 3, BQ - 1), 1)
        return 0

    lax.fori_loop(0, BQ // 2, _pair_body, 0)
    _wait(BQ - 1, 0)
    _wait(BQ - 1, 1)
    _out_wait(BQ - 2, 0)
    _out_wait(BQ - 1, 1)


@jax.jit
def _ges(qid, qcat, qbrand, match, id_tab, cat_tab, brand_tab, out_tab):
    mesh = plsc.VectorSubcoreMesh(core_axis_name="c", subcore_axis_name="s")
    kern = functools.partial(
        pl.kernel, mesh=mesh,
        out_type=jax.ShapeDtypeStruct((B * M,), jnp.float32),
        scratch_types=[
            pltpu.VMEM((3, BQ), jnp.int32),
            pltpu.VMEM((BQ * M,), jnp.int32),
            pltpu.VMEM((BQ, D), jnp.float32),
            pltpu.VMEM((BQ, D), jnp.float32),
            pltpu.VMEM((BQ, D), jnp.float32),
            pltpu.VMEM((MP, D), jnp.float32),
            pltpu.VMEM((MP, D), jnp.float32),
            pltpu.VMEM((MP,), jnp.float32),
            pltpu.VMEM((MP,), jnp.float32),
            pltpu.SemaphoreType.DMA,
            pltpu.SemaphoreType.DMA,
            pltpu.SemaphoreType.DMA,
            pltpu.SemaphoreType.DMA,
        ],
    )(_ges_kernel)
    out = kern(qid, qcat, qbrand, match, id_tab, cat_tab, brand_tab, out_tab)
    return out.reshape(B, M)


def kernel(query_item_id, query_cat_id, query_brand_id, match,
           id_table, cat_table, brand_table, out_table):
    qid = query_item_id.reshape(B).astype(jnp.int32)
    qcat = query_cat_id.reshape(B).astype(jnp.int32)
    qbrand = query_brand_id.reshape(B).astype(jnp.int32)
    return _ges(qid, qcat, qbrand, match.reshape(B * M).astype(jnp.int32),
                id_table, cat_table, brand_table, out_table)


# speedup vs baseline: 1.0036x; 1.0036x over previous
import functools

import jax
import jax.numpy as jnp
from jax import lax
from jax.experimental import pallas as pl
from jax.experimental.pallas import tpu as pltpu
from jax.experimental.pallas import tpu_sc as plsc

B = 4096
M = 200
D = 128
NC = 2
NS = 16
NW = NC * NS
BQ = B // NW
LANES = 16
DJ = D // LANES
MP = 208
MC = MP // LANES
MH0 = 104
MH1 = M - MH0


def _lane_perm(v, perm):
    return v.at[perm].get(mode="promise_in_bounds")


def _ges_kernel(qid_hbm, qcat_hbm, qbrand_hbm, match_hbm,
                id_tab, cat_tab, brand_tab, out_tab,
                out_hbm,
                qidx_v, midx_v, hid_v, rows0_v, rows1_v, logits_v,
                sem0, sem1, sem_out):
    wid = lax.axis_index("s") * NC + lax.axis_index("c")
    base = wid * BQ
    rows = [rows0_v, rows1_v]
    sems = [sem0, sem1]

    iota = lax.iota(jnp.int32, LANES)
    perms_p = [(iota + (1 << s)) & (LANES - 1) for s in range(4)]
    perms_m = [(iota + LANES - (1 << s)) & (LANES - 1) for s in range(4)]
    masks = [(iota % (2 << s)) < (1 << s) for s in range(4)]

    pltpu.sync_copy(match_hbm.at[pl.ds(base * M, BQ * M)], midx_v)

    pltpu.sync_copy(qid_hbm.at[pl.ds(base, BQ)], qidx_v.at[0])
    pltpu.sync_copy(qcat_hbm.at[pl.ds(base, BQ)], qidx_v.at[1])
    pltpu.sync_copy(qbrand_hbm.at[pl.ds(base, BQ)], qidx_v.at[2])
    cp0 = pltpu.async_copy(id_tab.at[qidx_v.at[0]], hid_v, sem0)
    cp1 = pltpu.async_copy(cat_tab.at[qidx_v.at[1]], rows0_v.at[pl.ds(0, BQ)], sem0)
    cp2 = pltpu.async_copy(brand_tab.at[qidx_v.at[2]], rows1_v.at[pl.ds(0, BQ)], sem0)
    cp0.wait()
    cp1.wait()
    cp2.wait()

    def _hid_body(i, _):
        q = i // DJ
        j = (i % DJ) * LANES
        hid_v[q, pl.ds(j, LANES)] = (
            hid_v[q, pl.ds(j, LANES)]
            + rows0_v[q, pl.ds(j, LANES)]
            + rows1_v[q, pl.ds(j, LANES)]
        ) * (1.0 / 3.0)
        return 0

    lax.fori_loop(0, BQ * DJ, _hid_body, 0)

    zeros = jnp.zeros((LANES,), jnp.float32)
    for rv in rows:
        for m in range(M, MP):
            for j in range(DJ):
                rv[m, pl.ds(j * LANES, LANES)] = zeros

    def _issue(q, slot):
        o0 = pl.multiple_of(q * M, 8)
        o1 = pl.multiple_of(q * M + MH0, 8)
        pltpu.async_copy(out_tab.at[midx_v.at[pl.ds(o0, MH0)]],
                         rows[slot].at[pl.ds(0, MH0)], sems[slot])
        pltpu.async_copy(out_tab.at[midx_v.at[pl.ds(o1, MH1)]],
                         rows[slot].at[pl.ds(MH0, MH1)], sems[slot])

    def _wait(q, slot):
        o0 = pl.multiple_of(q * M, 8)
        o1 = pl.multiple_of(q * M + MH0, 8)
        pltpu.make_async_copy(out_tab.at[midx_v.at[pl.ds(o0, MH0)]],
                              rows[slot].at[pl.ds(0, MH0)], sems[slot]).wait()
        pltpu.make_async_copy(out_tab.at[midx_v.at[pl.ds(o1, MH1)]],
                              rows[slot].at[pl.ds(MH0, MH1)], sems[slot]).wait()

    def _compute(q, slot):
        rv = rows[slot]
        hq = [hid_v[q, pl.ds(j * LANES, LANES)] for j in range(DJ)]

        def _chunk(mc, _):
            mb = mc * LANES
            accs = []
            for i in range(LANES):
                a = rv[mb + i, pl.ds(0, LANES)] * hq[0]
                for j in range(1, DJ):
                    a = a + rv[mb + i, pl.ds(j * LANES, LANES)] * hq[j]
                accs.append(a)
            cur = accs
            for s in range(4):
                nxt = []
                for k in range(len(cur) // 2):
                    a0, a1 = cur[2 * k], cur[2 * k + 1]
                    x = a0 + _lane_perm(a0, perms_p[s])
                    y = a1 + _lane_perm(a1, perms_m[s])
                    nxt.append(jnp.where(masks[s], x, y))
                cur = nxt
            logits_v[pl.ds(pl.multiple_of(q * MP + mb, 16), LANES)] = cur[0]
            return 0

        lax.fori_loop(0, MC, _chunk, 0)
        pltpu.async_copy(logits_v.at[pl.ds(pl.multiple_of(q * MP, 16), M)],
                         out_hbm.at[pl.ds(pl.multiple_of((base + q) * M, 8), M)],
                         sem_out)

    _issue(0, 0)

    def _pair_body(i, _):
        q0 = 2 * i
        _issue(q0 + 1, 1)
        _wait(q0, 0)
        _compute(q0, 0)
        _issue(jnp.minimum(q0 + 2, BQ - 1), 0)
        _wait(q0 + 1, 1)
        _compute(q0 + 1, 1)
        return 0

    lax.fori_loop(0, BQ // 2, _pair_body, 0)
    _wait(BQ - 1, 0)

    def _drain(q, _):
        pltpu.make_async_copy(logits_v.at[pl.ds(pl.multiple_of(q * MP, 16), M)],
                              out_hbm.at[pl.ds(pl.multiple_of((base + q) * M, 8), M)],
                              sem_out).wait()
        return 0

    lax.fori_loop(0, BQ, _drain, 0)


@jax.jit
def _ges(qid, qcat, qbrand, match, id_tab, cat_tab, brand_tab, out_tab):
    mesh = plsc.VectorSubcoreMesh(core_axis_name="c", subcore_axis_name="s")
    kern = functools.partial(
        pl.kernel, mesh=mesh,
        out_type=jax.ShapeDtypeStruct((B * M,), jnp.float32),
        scratch_types=[
            pltpu.VMEM((3, BQ), jnp.int32),
            pltpu.VMEM((BQ * M,), jnp.int32),
            pltpu.VMEM((BQ, D), jnp.float32),
            pltpu.VMEM((MP, D), jnp.float32),
            pltpu.VMEM((MP, D), jnp.float32),
            pltpu.VMEM((BQ * MP,), jnp.float32),
            pltpu.SemaphoreType.DMA,
            pltpu.SemaphoreType.DMA,
            pltpu.SemaphoreType.DMA,
        ],
    )(_ges_kernel)
    out = kern(qid, qcat, qbrand, match, id_tab, cat_tab, brand_tab, out_tab)
    return out.reshape(B, M)


def kernel(query_item_id, query_cat_id, query_brand_id, match,
           id_table, cat_table, brand_table, out_table):
    qid = query_item_id.reshape(B).astype(jnp.int32)
    qcat = query_cat_id.reshape(B).astype(jnp.int32)
    qbrand = query_brand_id.reshape(B).astype(jnp.int32)
    return _ges(qid, qcat, qbrand, match.reshape(B * M).astype(jnp.int32),
                id_table, cat_table, brand_table, out_table)


# speedup vs baseline: 1.0230x; 1.0193x over previous
import functools

import jax
import jax.numpy as jnp
from jax import lax
from jax.experimental import pallas as pl
from jax.experimental.pallas import tpu as pltpu
from jax.experimental.pallas import tpu_sc as plsc

B = 4096
M = 200
D = 128
NC = 2
NS = 16
NW = NC * NS
BQ = B // NW
LANES = 16
DJ = D // LANES
MP = 208
MC = MP // LANES
MH0 = 104
MH1 = M - MH0


def _lane_perm(v, perm):
    return v.at[perm].get(mode="promise_in_bounds")


def _ges_kernel(qid_hbm, qcat_hbm, qbrand_hbm, match_hbm,
                id_tab, cat_tab, brand_tab, out_tab,
                out_hbm,
                qidx_v, midx_v, hid_v, rows0_v, rows1_v, logits_v,
                sem0, sem1, sem_out):
    wid = lax.axis_index("s") * NC + lax.axis_index("c")
    base = wid * BQ
    rows = [rows0_v, rows1_v]
    sems = [sem0, sem1]

    iota = lax.iota(jnp.int32, LANES)
    perms_p = [(iota + (1 << s)) & (LANES - 1) for s in range(4)]
    perms_m = [(iota + LANES - (1 << s)) & (LANES - 1) for s in range(4)]
    masks = [(iota % (2 << s)) < (1 << s) for s in range(4)]

    def _issue(q, slot):
        o0 = pl.multiple_of(q * M, 8)
        o1 = pl.multiple_of(q * M + MH0, 8)
        pltpu.async_copy(out_tab.at[midx_v.at[pl.ds(o0, MH0)]],
                         rows[slot].at[pl.ds(0, MH0)], sems[slot])
        pltpu.async_copy(out_tab.at[midx_v.at[pl.ds(o1, MH1)]],
                         rows[slot].at[pl.ds(MH0, MH1)], sems[slot])

    def _wait(q, slot):
        o0 = pl.multiple_of(q * M, 8)
        o1 = pl.multiple_of(q * M + MH0, 8)
        pltpu.make_async_copy(out_tab.at[midx_v.at[pl.ds(o0, MH0)]],
                              rows[slot].at[pl.ds(0, MH0)], sems[slot]).wait()
        pltpu.make_async_copy(out_tab.at[midx_v.at[pl.ds(o1, MH1)]],
                              rows[slot].at[pl.ds(MH0, MH1)], sems[slot]).wait()

    cp_m = pltpu.async_copy(match_hbm.at[pl.ds(base * M, BQ * M)], midx_v,
                            sem_out)
    pltpu.sync_copy(qid_hbm.at[pl.ds(base, BQ)], qidx_v.at[0])
    pltpu.sync_copy(qcat_hbm.at[pl.ds(base, BQ)], qidx_v.at[1])
    pltpu.sync_copy(qbrand_hbm.at[pl.ds(base, BQ)], qidx_v.at[2])
    cp0 = pltpu.async_copy(id_tab.at[qidx_v.at[0]], hid_v, sem0)
    cp1 = pltpu.async_copy(cat_tab.at[qidx_v.at[1]], rows0_v.at[pl.ds(0, BQ)],
                           sem0)
    cp_m.wait()

    zeros = jnp.zeros((LANES,), jnp.float32)
    for m in range(M, MP):
        for j in range(DJ):
            rows1_v[m, pl.ds(j * LANES, LANES)] = zeros
    _issue(0, 1)

    cp0.wait()
    cp1.wait()

    def _hid_add(q, _):
        for j in range(DJ):
            sl = pl.ds(j * LANES, LANES)
            hid_v[q, sl] = hid_v[q, sl] + rows0_v[q, sl]
        return 0

    lax.fori_loop(0, BQ, _hid_add, 0)

    cp2 = pltpu.async_copy(brand_tab.at[qidx_v.at[2]], rows0_v.at[pl.ds(0, BQ)],
                           sem0)
    cp2.wait()

    def _hid_fin(q, _):
        for j in range(DJ):
            sl = pl.ds(j * LANES, LANES)
            hid_v[q, sl] = (hid_v[q, sl] + rows0_v[q, sl]) * (1.0 / 3.0)
        return 0

    lax.fori_loop(0, BQ, _hid_fin, 0)

    for m in range(M, MP):
        for j in range(DJ):
            rows0_v[m, pl.ds(j * LANES, LANES)] = zeros
    _issue(1, 0)

    def _compute(q, slot):
        rv = rows[slot]
        hq = [hid_v[q, pl.ds(j * LANES, LANES)] for j in range(DJ)]

        def _chunk(mc, _):
            mb = mc * LANES
            accs = []
            for i in range(LANES):
                a = rv[mb + i, pl.ds(0, LANES)] * hq[0]
                for j in range(1, DJ):
                    a = a + rv[mb + i, pl.ds(j * LANES, LANES)] * hq[j]
                accs.append(a)
            cur = accs
            for s in range(4):
                nxt = []
                for k in range(len(cur) // 2):
                    a0, a1 = cur[2 * k], cur[2 * k + 1]
                    x = a0 + _lane_perm(a0, perms_p[s])
                    y = a1 + _lane_perm(a1, perms_m[s])
                    nxt.append(jnp.where(masks[s], x, y))
                cur = nxt
            logits_v[pl.ds(pl.multiple_of(q * MP + mb, 16), LANES)] = cur[0]
            return 0

        lax.fori_loop(0, MC, _chunk, 0)
        pltpu.async_copy(logits_v.at[pl.ds(pl.multiple_of(q * MP, 16), M)],
                         out_hbm.at[pl.ds(pl.multiple_of((base + q) * M, 8), M)],
                         sem_out)

    def _pair_body(i, _):
        q0 = 2 * i
        _wait(q0, 1)
        _compute(q0, 1)
        _issue(q0 + 2, 1)
        _wait(q0 + 1, 0)
        _compute(q0 + 1, 0)
        _issue(q0 + 3, 0)
        return 0

    lax.fori_loop(0, BQ // 2 - 1, _pair_body, 0)
    _wait(BQ - 2, 1)
    _compute(BQ - 2, 1)
    _wait(BQ - 1, 0)
    _compute(BQ - 1, 0)

    def _drain(q, _):
        pltpu.make_async_copy(logits_v.at[pl.ds(pl.multiple_of(q * MP, 16), M)],
                              out_hbm.at[pl.ds(pl.multiple_of((base + q) * M, 8), M)],
                              sem_out).wait()
        return 0

    lax.fori_loop(0, BQ, _drain, 0)


@jax.jit
def _ges(qid, qcat, qbrand, match, id_tab, cat_tab, brand_tab, out_tab):
    mesh = plsc.VectorSubcoreMesh(core_axis_name="c", subcore_axis_name="s")
    kern = functools.partial(
        pl.kernel, mesh=mesh,
        out_type=jax.ShapeDtypeStruct((B * M,), jnp.float32),
        scratch_types=[
            pltpu.VMEM((3, BQ), jnp.int32),
            pltpu.VMEM((BQ * M,), jnp.int32),
            pltpu.VMEM((BQ, D), jnp.float32),
            pltpu.VMEM((MP, D), jnp.float32),
            pltpu.VMEM((MP, D), jnp.float32),
            pltpu.VMEM((BQ * MP,), jnp.float32),
            pltpu.SemaphoreType.DMA,
            pltpu.SemaphoreType.DMA,
            pltpu.SemaphoreType.DMA,
        ],
    )(_ges_kernel)
    out = kern(qid, qcat, qbrand, match, id_tab, cat_tab, brand_tab, out_tab)
    return out.reshape(B, M)


def kernel(query_item_id, query_cat_id, query_brand_id, match,
           id_table, cat_table, brand_table, out_table):
    qid = query_item_id.reshape(B).astype(jnp.int32)
    qcat = query_cat_id.reshape(B).astype(jnp.int32)
    qbrand = query_brand_id.reshape(B).astype(jnp.int32)
    return _ges(qid, qcat, qbrand, match.reshape(B * M).astype(jnp.int32),
                id_table, cat_table, brand_table, out_table)
